# baseline (speedup 1.0000x reference)
import functools

import jax
import jax.numpy as jnp
from jax import lax
from jax.experimental import pallas as pl
from jax.experimental.pallas import tpu as pltpu
from jax.experimental.pallas import tpu_sc as plsc

B = 4096
F = 20
V = 1000000
D = 64
H = 512
A = 4

NC = 2
NS = 16
NW = NC * NS
BPW = B // NW
SUB = 16
NSUB = BPW // SUB
RPS = SUB * F
GCH = 64
NG = RPS // GCH
LANES = 16
DV = D // LANES
PD = 2 * D


def _pool_body(idx_hbm, pair_hbm, table_hbm, out_hbm, idx_v, pair_v, rows_v,
               pooled_v, sem):
    wid = lax.axis_index("s") * NC + lax.axis_index("c")
    base_row = wid * BPW

    pltpu.sync_copy(idx_hbm.at[pl.ds(base_row * F, BPW * F)], idx_v)
    pltpu.sync_copy(pair_hbm.at[pl.ds(base_row * F, BPW * F)], pair_v)

    def fire(sc, buf):
        handles = []
        for g in range(NG):
            h = pltpu.async_copy(
                table_hbm.at[pair_v.at[pl.ds(sc * RPS + g * GCH, GCH)]],
                rows_v.at[buf].at[pl.ds(g * GCH, GCH)],
                sem,
            )
            handles.append(h)
        return handles

    iota = lax.iota(jnp.int32, LANES)
    buf_idx = [jnp.full((LANES,), k, jnp.int32) for k in range(2)]

    def pool_and_store(sc, buf):
        def body_b(b, _):
            accs = [jnp.zeros((LANES,), jnp.float32) for _ in range(DV)]
            for f in range(F):
                j = b * F + f
                jsplat = jnp.full((LANES,), 0, jnp.int32) + j
                gsplat = jsplat + sc * RPS
                idxspl = plsc.load_gather(idx_v, [gsplat])
                col0 = (idxspl & 1) * D + iota
                for c in range(DV):
                    val = plsc.load_gather(
                        rows_v, [buf_idx[buf], jsplat, col0 + c * LANES])
                    accs[c] = accs[c] + val
            for c in range(DV):
                pooled_v[b, pl.ds(c * LANES, LANES)] = accs[c]
            return 0

        lax.fori_loop(0, SUB, body_b, 0)
        pltpu.sync_copy(pooled_v, out_hbm.at[pl.ds(base_row + sc * SUB, SUB)])

    pending = fire(0, 0)
    for sc in range(NSUB):
        buf = sc % 2
        for h in pending:
            h.wait()
        if sc + 1 < NSUB:
            pending = fire(sc + 1, (sc + 1) % 2)
        pool_and_store(sc, buf)


@jax.jit
def _gather_pool(idx_flat, pair_flat, table_pairs):
    mesh = plsc.VectorSubcoreMesh(core_axis_name="c", subcore_axis_name="s")
    kern = functools.partial(
        pl.kernel,
        out_type=jax.ShapeDtypeStruct((B, D), jnp.float32),
        mesh=mesh,
        scratch_types=[
            pltpu.VMEM((BPW * F,), jnp.int32),
            pltpu.VMEM((BPW * F,), jnp.int32),
            pltpu.VMEM((2, RPS, PD), jnp.float32),
            pltpu.VMEM((SUB, D), jnp.float32),
            pltpu.SemaphoreType.DMA,
        ],
        compiler_params=pltpu.CompilerParams(needs_layout_passes=False),
    )(_pool_body)
    return kern(idx_flat, pair_flat, table_pairs)


def _mlp_body(p_ref, w1_ref, b1_ref, w2_ref, b2_ref, y_ref):
    h = jnp.dot(p_ref[...], w1_ref[...], preferred_element_type=jnp.float32)
    h = jnp.maximum(h + b1_ref[...], 0.0)
    y_ref[...] = jnp.dot(h, w2_ref[...], preferred_element_type=jnp.float32) + b2_ref[...]


MLP_BLK = 1024


def _mlp(pooled, W1, b1, W2, b2):
    return pl.pallas_call(
        _mlp_body,
        grid=(B // MLP_BLK,),
        in_specs=[
            pl.BlockSpec((MLP_BLK, D), lambda i: (i, 0)),
            pl.BlockSpec((D, H), lambda i: (0, 0)),
            pl.BlockSpec((1, H), lambda i: (0, 0)),
            pl.BlockSpec((H, A), lambda i: (0, 0)),
            pl.BlockSpec((1, A), lambda i: (0, 0)),
        ],
        out_specs=pl.BlockSpec((MLP_BLK, A), lambda i: (i, 0)),
        out_shape=jax.ShapeDtypeStruct((B, A), jnp.float32),
    )(pooled, W1, b1, W2, b2)


def kernel(indices, table, W1, b1, W2, b2):
    idx_flat = indices.reshape(-1).astype(jnp.int32)
    pair_flat = idx_flat >> 1
    pooled = _gather_pool(idx_flat, pair_flat, table.reshape(V // 2, PD))
    return _mlp(pooled, W1, b1.reshape(1, H), W2, b2.reshape(1, A))


# speedup vs baseline: 1.1057x; 1.1057x over previous
import functools

import jax
import jax.numpy as jnp
from jax import lax
from jax.experimental import pallas as pl
from jax.experimental.pallas import tpu as pltpu
from jax.experimental.pallas import tpu_sc as plsc

B = 4096
F = 20
V = 1000000
D = 64
H = 512
A = 4

NC = 2
NS = 16
NW = NC * NS
BPW = B // NW
SUB = 16
NSUB = BPW // SUB
RPS = SUB * F
GCH = 64
NG = RPS // GCH
LANES = 16
DV = D // LANES
PD = 128


def _pool_body(idx_hbm, table_hbm, out_hbm, idx_v, rows_v, pooled_v, sem):
    wid = lax.axis_index("s") * NC + lax.axis_index("c")
    base_row = wid * BPW

    pltpu.sync_copy(idx_hbm.at[pl.ds(base_row * F, BPW * F)], idx_v)

    def fire(sc, buf):
        handles = []
        for g in range(NG):
            h = pltpu.async_copy(
                table_hbm.at[idx_v.at[pl.ds(sc * RPS + g * GCH, GCH)]],
                rows_v.at[buf].at[pl.ds(g * GCH, GCH)],
                sem,
            )
            handles.append(h)
        return handles

    def pool_and_store(sc, buf):
        rows = rows_v.at[buf]

        def body_b(b, _):
            r0 = b * F
            for c in range(DV):
                acc = rows[r0, pl.ds(c * LANES, LANES)]
                for f in range(1, F):
                    acc = acc + rows[r0 + f, pl.ds(c * LANES, LANES)]
                pooled_v[b, pl.ds(c * LANES, LANES)] = acc
            return 0

        lax.fori_loop(0, SUB, body_b, 0)
        pltpu.sync_copy(pooled_v, out_hbm.at[pl.ds(base_row + sc * SUB, SUB)])

    pending = fire(0, 0)
    for sc in range(NSUB):
        buf = sc % 2
        for h in pending:
            h.wait()
        if sc + 1 < NSUB:
            pending = fire(sc + 1, (sc + 1) % 2)
        pool_and_store(sc, buf)


@jax.jit
def _gather_pool(idx_flat, table_wide):
    mesh = plsc.VectorSubcoreMesh(core_axis_name="c", subcore_axis_name="s")
    kern = functools.partial(
        pl.kernel,
        out_type=jax.ShapeDtypeStruct((B, D), jnp.float32),
        mesh=mesh,
        scratch_types=[
            pltpu.VMEM((BPW * F,), jnp.int32),
            pltpu.VMEM((2, RPS, PD), jnp.float32),
            pltpu.VMEM((SUB, D), jnp.float32),
            pltpu.SemaphoreType.DMA,
        ],
        compiler_params=pltpu.CompilerParams(needs_layout_passes=False),
    )(_pool_body)
    return kern(idx_flat, table_wide)


def _mlp_body(p_ref, w1_ref, b1_ref, w2_ref, b2_ref, y_ref):
    h = jnp.dot(p_ref[...], w1_ref[...], preferred_element_type=jnp.float32)
    h = jnp.maximum(h + b1_ref[...], 0.0)
    y_ref[...] = jnp.dot(h, w2_ref[...], preferred_element_type=jnp.float32) + b2_ref[...]


MLP_BLK = 1024


def _mlp(pooled, W1, b1, W2, b2):
    return pl.pallas_call(
        _mlp_body,
        grid=(B // MLP_BLK,),
        in_specs=[
            pl.BlockSpec((MLP_BLK, D), lambda i: (i, 0)),
            pl.BlockSpec((D, H), lambda i: (0, 0)),
            pl.BlockSpec((1, H), lambda i: (0, 0)),
            pl.BlockSpec((H, A), lambda i: (0, 0)),
            pl.BlockSpec((1, A), lambda i: (0, 0)),
        ],
        out_specs=pl.BlockSpec((MLP_BLK, A), lambda i: (i, 0)),
        out_shape=jax.ShapeDtypeStruct((B, A), jnp.float32),
    )(pooled, W1, b1, W2, b2)


def kernel(indices, table, W1, b1, W2, b2):
    idx_flat = indices.reshape(-1).astype(jnp.int32)
    table_wide = jnp.pad(table, ((0, 0), (0, PD - D)))
    pooled = _gather_pool(idx_flat, table_wide)
    return _mlp(pooled, W1, b1.reshape(1, H), W2, b2.reshape(1, A))


# speedup vs baseline: 1.6609x; 1.5021x over previous
import functools

import jax
import jax.numpy as jnp
from jax import lax
from jax.experimental import pallas as pl
from jax.experimental.pallas import tpu as pltpu
from jax.experimental.pallas import tpu_sc as plsc

B = 4096
F = 20
V = 1000000
D = 64
H = 512
A = 4

NC = 2
NS = 16
NW = NC * NS
BPW = B // NW
SUB = 16
NSUB = BPW // SUB
RPS = SUB * F
GCH = 64
NG = RPS // GCH
LANES = 16
DV = D // LANES
PD = 128


def _pool_body(idx_hbm, table_hbm, out_hbm, idx_v, rows_v, pooled_v, sem):
    wid = lax.axis_index("s") * NC + lax.axis_index("c")
    base_row = wid * BPW

    pltpu.sync_copy(idx_hbm.at[pl.ds(base_row * F, BPW * F)], idx_v)

    def fire(sc, buf):
        handles = []
        for g in range(NG):
            h = pltpu.async_copy(
                table_hbm.at[idx_v.at[pl.ds(sc * RPS + g * GCH, GCH)]],
                rows_v.at[buf].at[pl.ds(g * GCH, GCH)],
                sem,
            )
            handles.append(h)
        return handles

    def pool_and_store(sc, buf):
        rows = rows_v.at[buf]

        def body_b(b, _):
            r0 = b * F
            for c in range(DV):
                acc = rows[r0, pl.ds(c * LANES, LANES)]
                for f in range(1, F):
                    acc = acc + rows[r0 + f, pl.ds(c * LANES, LANES)]
                pooled_v[b, pl.ds(c * LANES, LANES)] = acc
            return 0

        lax.fori_loop(0, SUB, body_b, 0)
        pltpu.sync_copy(pooled_v, out_hbm.at[pl.ds(base_row + sc * SUB, SUB)])

    pending = fire(0, 0)
    for sc in range(NSUB):
        buf = sc % 2
        for h in pending:
            h.wait()
        if sc + 1 < NSUB:
            pending = fire(sc + 1, (sc + 1) % 2)
        pool_and_store(sc, buf)


@jax.jit
def _gather_pool(idx_flat, table_wide):
    mesh = plsc.VectorSubcoreMesh(core_axis_name="c", subcore_axis_name="s")
    kern = functools.partial(
        pl.kernel,
        out_type=jax.ShapeDtypeStruct((B, D), jnp.float32),
        mesh=mesh,
        scratch_types=[
            pltpu.VMEM((BPW * F,), jnp.int32),
            pltpu.VMEM((2, RPS, PD), jnp.float32),
            pltpu.VMEM((SUB, D), jnp.float32),
            pltpu.SemaphoreType.DMA,
        ],
        compiler_params=pltpu.CompilerParams(needs_layout_passes=False),
    )(_pool_body)
    return kern(idx_flat, table_wide)


TCB = 4096


def _tr_body(t_ref, o_ref):
    x = t_ref[...]
    xt = x.T
    z = jnp.zeros((TCB, PD - D), jnp.float32)
    o_ref[...] = jnp.concatenate([xt, z], axis=1)


def _transpose_pad(tableT):
    return pl.pallas_call(
        _tr_body,
        grid=(pl.cdiv(V, TCB),),
        in_specs=[pl.BlockSpec((D, TCB), lambda i: (0, i))],
        out_specs=pl.BlockSpec((TCB, PD), lambda i: (i, 0)),
        out_shape=jax.ShapeDtypeStruct((V, PD), jnp.float32),
    )(tableT)


def _mlp_body(p_ref, w1_ref, b1_ref, w2_ref, b2_ref, y_ref):
    h = jnp.dot(p_ref[...], w1_ref[...], preferred_element_type=jnp.float32)
    h = jnp.maximum(h + b1_ref[...], 0.0)
    y_ref[...] = jnp.dot(h, w2_ref[...], preferred_element_type=jnp.float32) + b2_ref[...]


MLP_BLK = 1024


def _mlp(pooled, W1, b1, W2, b2):
    return pl.pallas_call(
        _mlp_body,
        grid=(B // MLP_BLK,),
        in_specs=[
            pl.BlockSpec((MLP_BLK, D), lambda i: (i, 0)),
            pl.BlockSpec((D, H), lambda i: (0, 0)),
            pl.BlockSpec((1, H), lambda i: (0, 0)),
            pl.BlockSpec((H, A), lambda i: (0, 0)),
            pl.BlockSpec((1, A), lambda i: (0, 0)),
        ],
        out_specs=pl.BlockSpec((MLP_BLK, A), lambda i: (i, 0)),
        out_shape=jax.ShapeDtypeStruct((B, A), jnp.float32),
    )(pooled, W1, b1, W2, b2)


def kernel(indices, table, W1, b1, W2, b2):
    idx_flat = indices.reshape(-1).astype(jnp.int32)
    table_wide = _transpose_pad(table.T)
    pooled = _gather_pool(idx_flat, table_wide)
    return _mlp(pooled, W1, b1.reshape(1, H), W2, b2.reshape(1, A))


# speedup vs baseline: 1.7158x; 1.0330x over previous
import functools

import jax
import jax.numpy as jnp
from jax import lax
from jax.experimental import pallas as pl
from jax.experimental.pallas import tpu as pltpu
from jax.experimental.pallas import tpu_sc as plsc

B = 4096
F = 20
V = 1000000
D = 64
H = 512
A = 4

NC = 2
NS = 16
NW = NC * NS
BPW = B // NW
SUB = 16
NSUB = BPW // SUB
RPS = SUB * F
GCH = 64
NG = RPS // GCH
LANES = 16
DV = D // LANES
PD = 2 * D


def _pool_body(idx_hbm, pair_hbm, table_hbm, out_hbm, idx_v, pair_v, rows_v,
               pooled_v, sem):
    wid = lax.axis_index("s") * NC + lax.axis_index("c")
    base_row = wid * BPW

    pltpu.sync_copy(idx_hbm.at[pl.ds(base_row * F, BPW * F)], idx_v)
    pltpu.sync_copy(pair_hbm.at[pl.ds(base_row * F, BPW * F)], pair_v)

    def fire(sc, buf):
        handles = []
        for g in range(NG):
            h = pltpu.async_copy(
                table_hbm.at[pair_v.at[pl.ds(sc * RPS + g * GCH, GCH)]],
                rows_v.at[buf].at[pl.ds(g * GCH, GCH)],
                sem,
            )
            handles.append(h)
        return handles

    iota = lax.iota(jnp.int32, LANES)
    buf_idx = [jnp.full((LANES,), k, jnp.int32) for k in range(2)]

    def pool_and_store(sc, buf):
        def body_b(b, _):
            accs = [jnp.zeros((LANES,), jnp.float32) for _ in range(DV)]
            for f in range(F):
                j = b * F + f
                jsplat = jnp.full((LANES,), 0, jnp.int32) + j
                gsplat = jsplat + sc * RPS
                idxspl = plsc.load_gather(idx_v, [gsplat])
                col0 = ((idxspl >> 11) & 1) * D + iota
                for c in range(DV):
                    val = plsc.load_gather(
                        rows_v, [buf_idx[buf], jsplat, col0 + c * LANES])
                    accs[c] = accs[c] + val
            for c in range(DV):
                pooled_v[b, pl.ds(c * LANES, LANES)] = accs[c]
            return 0

        lax.fori_loop(0, SUB, body_b, 0)
        pltpu.sync_copy(pooled_v, out_hbm.at[pl.ds(base_row + sc * SUB, SUB)])

    pending = fire(0, 0)
    for sc in range(NSUB):
        buf = sc % 2
        for h in pending:
            h.wait()
        if sc + 1 < NSUB:
            pending = fire(sc + 1, (sc + 1) % 2)
        pool_and_store(sc, buf)


@jax.jit
def _gather_pool(idx_flat, pair_flat, table_pairs):
    mesh = plsc.VectorSubcoreMesh(core_axis_name="c", subcore_axis_name="s")
    kern = functools.partial(
        pl.kernel,
        out_type=jax.ShapeDtypeStruct((B, D), jnp.float32),
        mesh=mesh,
        scratch_types=[
            pltpu.VMEM((BPW * F,), jnp.int32),
            pltpu.VMEM((BPW * F,), jnp.int32),
            pltpu.VMEM((2, RPS, PD), jnp.float32),
            pltpu.VMEM((SUB, D), jnp.float32),
            pltpu.SemaphoreType.DMA,
        ],
        compiler_params=pltpu.CompilerParams(needs_layout_passes=False),
    )(_pool_body)
    return kern(idx_flat, pair_flat, table_pairs)


TCB = 4096
HB = TCB // 2
NTB = -(-V // TCB)
VP = NTB * HB


def _tr_body(t_ref, o_ref):
    x = t_ref[...]
    lo = x[:, :HB].T
    hi = x[:, HB:].T
    o_ref[...] = jnp.concatenate([lo, hi], axis=1)


def _transpose_pairs(tableT):
    return pl.pallas_call(
        _tr_body,
        grid=(NTB,),
        in_specs=[pl.BlockSpec((D, TCB), lambda i: (0, i))],
        out_specs=pl.BlockSpec((HB, PD), lambda i: (i, 0)),
        out_shape=jax.ShapeDtypeStruct((VP, PD), jnp.float32),
    )(tableT)


def _mlp_body(p_ref, w1_ref, b1_ref, w2_ref, b2_ref, y_ref):
    h = jnp.dot(p_ref[...], w1_ref[...], preferred_element_type=jnp.float32)
    h = jnp.maximum(h + b1_ref[...], 0.0)
    y_ref[...] = jnp.dot(h, w2_ref[...], preferred_element_type=jnp.float32) + b2_ref[...]


MLP_BLK = 1024


def _mlp(pooled, W1, b1, W2, b2):
    return pl.pallas_call(
        _mlp_body,
        grid=(B // MLP_BLK,),
        in_specs=[
            pl.BlockSpec((MLP_BLK, D), lambda i: (i, 0)),
            pl.BlockSpec((D, H), lambda i: (0, 0)),
            pl.BlockSpec((1, H), lambda i: (0, 0)),
            pl.BlockSpec((H, A), lambda i: (0, 0)),
            pl.BlockSpec((1, A), lambda i: (0, 0)),
        ],
        out_specs=pl.BlockSpec((MLP_BLK, A), lambda i: (i, 0)),
        out_shape=jax.ShapeDtypeStruct((B, A), jnp.float32),
    )(pooled, W1, b1, W2, b2)


def kernel(indices, table, W1, b1, W2, b2):
    idx_flat = indices.reshape(-1).astype(jnp.int32)
    pair_flat = ((idx_flat >> 12) << 11) | (idx_flat & (HB - 1))
    table_pairs = _transpose_pairs(table.T)
    pooled = _gather_pool(idx_flat, pair_flat, table_pairs)
    return _mlp(pooled, W1, b1.reshape(1, H), W2, b2.reshape(1, A))


# speedup vs baseline: 2.0221x; 1.1785x over previous
import functools

import jax
import jax.numpy as jnp
from jax import lax
from jax.experimental import pallas as pl
from jax.experimental.pallas import tpu as pltpu
from jax.experimental.pallas import tpu_sc as plsc

B = 4096
F = 20
V = 1000000
D = 64
H = 512
A = 4

NC = 2
NS = 16
NW = NC * NS
BPW = B // NW
SUB = 16
NSUB = BPW // SUB
RPS = SUB * F
GCH = 64
NG = RPS // GCH
LANES = 16
DV = D // LANES
PD = 2 * D


def _pool_body(idx_hbm, pair_hbm, table_hbm, out_hbm, idx_v, pair_v, rows_v,
               pooled_v, sem):
    wid = lax.axis_index("s") * NC + lax.axis_index("c")
    base_row = wid * BPW

    pltpu.sync_copy(idx_hbm.at[pl.ds(base_row * F, BPW * F)], idx_v)
    pltpu.sync_copy(pair_hbm.at[pl.ds(base_row * F, BPW * F)], pair_v)

    def fire(sc, buf):
        handles = []
        for g in range(NG):
            h = pltpu.async_copy(
                table_hbm.at[pair_v.at[pl.ds(sc * RPS + g * GCH, GCH)]],
                rows_v.at[buf].at[pl.ds(g * GCH, GCH)],
                sem,
            )
            handles.append(h)
        return handles

    iota = lax.iota(jnp.int32, LANES)
    buf_idx = [jnp.full((LANES,), k, jnp.int32) for k in range(2)]

    def pool_and_store(sc, buf):
        def body_b(b, _):
            accs = [jnp.zeros((LANES,), jnp.float32) for _ in range(DV)]
            for f in range(F):
                j = b * F + f
                jsplat = jnp.full((LANES,), 0, jnp.int32) + j
                gsplat = jsplat + sc * RPS
                idxspl = plsc.load_gather(idx_v, [gsplat])
                col0 = ((idxspl >> 11) & 1) * D + iota
                for c in range(DV):
                    val = plsc.load_gather(
                        rows_v, [buf_idx[buf], jsplat, col0 + c * LANES])
                    accs[c] = accs[c] + val
            for c in range(DV):
                pooled_v[b, pl.ds(c * LANES, LANES)] = accs[c]
            return 0

        lax.fori_loop(0, SUB, body_b, 0)
        pltpu.sync_copy(pooled_v, out_hbm.at[pl.ds(base_row + sc * SUB, SUB)])

    pending = fire(0, 0)
    for sc in range(NSUB):
        buf = sc % 2
        for h in pending:
            h.wait()
        if sc + 1 < NSUB:
            pending = fire(sc + 1, (sc + 1) % 2)
        pool_and_store(sc, buf)


@jax.jit
def _gather_pool(idx_flat, pair_flat, table_pairs):
    mesh = plsc.VectorSubcoreMesh(core_axis_name="c", subcore_axis_name="s")
    kern = functools.partial(
        pl.kernel,
        out_type=jax.ShapeDtypeStruct((B, D), jnp.float32),
        mesh=mesh,
        scratch_types=[
            pltpu.VMEM((BPW * F,), jnp.int32),
            pltpu.VMEM((BPW * F,), jnp.int32),
            pltpu.VMEM((2, RPS, PD), jnp.float32),
            pltpu.VMEM((SUB, D), jnp.float32),
            pltpu.SemaphoreType.DMA,
        ],
        compiler_params=pltpu.CompilerParams(needs_layout_passes=False),
    )(_pool_body)
    return kern(idx_flat, pair_flat, table_pairs)


TCB = 4096
HB = TCB // 2
NTB = -(-V // TCB)
VP = NTB * HB


def _tr_body(t_ref, o_ref):
    x = t_ref[...]
    xs = jnp.concatenate([x[:, :HB], x[:, HB:]], axis=0)
    o_ref[...] = xs.T


def _transpose_pairs(tableT):
    return pl.pallas_call(
        _tr_body,
        grid=(NTB,),
        in_specs=[pl.BlockSpec((D, TCB), lambda i: (0, i))],
        out_specs=pl.BlockSpec((HB, PD), lambda i: (i, 0)),
        out_shape=jax.ShapeDtypeStruct((VP, PD), jnp.float32),
    )(tableT)


def _mlp_body(p_ref, w1_ref, b1_ref, w2_ref, b2_ref, y_ref):
    h = jnp.dot(p_ref[...], w1_ref[...], preferred_element_type=jnp.float32)
    h = jnp.maximum(h + b1_ref[...], 0.0)
    y_ref[...] = jnp.dot(h, w2_ref[...], preferred_element_type=jnp.float32) + b2_ref[...]


MLP_BLK = 1024


def _mlp(pooled, W1, b1, W2, b2):
    return pl.pallas_call(
        _mlp_body,
        grid=(B // MLP_BLK,),
        in_specs=[
            pl.BlockSpec((MLP_BLK, D), lambda i: (i, 0)),
            pl.BlockSpec((D, H), lambda i: (0, 0)),
            pl.BlockSpec((1, H), lambda i: (0, 0)),
            pl.BlockSpec((H, A), lambda i: (0, 0)),
            pl.BlockSpec((1, A), lambda i: (0, 0)),
        ],
        out_specs=pl.BlockSpec((MLP_BLK, A), lambda i: (i, 0)),
        out_shape=jax.ShapeDtypeStruct((B, A), jnp.float32),
    )(pooled, W1, b1, W2, b2)


def kernel(indices, table, W1, b1, W2, b2):
    idx_flat = indices.reshape(-1).astype(jnp.int32)
    pair_flat = ((idx_flat >> 12) << 11) | (idx_flat & (HB - 1))
    table_pairs = _transpose_pairs(table.T)
    pooled = _gather_pool(idx_flat, pair_flat, table_pairs)
    return _mlp(pooled, W1, b1.reshape(1, H), W2, b2.reshape(1, A))


# speedup vs baseline: 2.3210x; 1.1478x over previous
import functools

import jax
import jax.numpy as jnp
from jax import lax
from jax.experimental import pallas as pl
from jax.experimental.pallas import tpu as pltpu
from jax.experimental.pallas import tpu_sc as plsc

B = 4096
F = 20
V = 1000000
D = 64
H = 512
A = 4

NC = 2
NS = 16
NW = NC * NS
BPW = B // NW
SUB = 16
NSUB = BPW // SUB
RPS = SUB * F
GCH = 64
NG = RPS // GCH
LANES = 16
DV = D // LANES
PD = 128

TCB = 4096
QB = TCB // 4
NTB = -(-V // TCB)
VP = NTB * QB


def _pool_body(idx_hbm, quad_hbm, table_hbm, out_hbm, idx_v, quad_v, rows_v,
               pooled_v, sem):
    wid = lax.axis_index("s") * NC + lax.axis_index("c")
    base_row = wid * BPW

    pltpu.sync_copy(idx_hbm.at[pl.ds(base_row * F, BPW * F)], idx_v)
    pltpu.sync_copy(quad_hbm.at[pl.ds(base_row * F, BPW * F)], quad_v)

    def fire(sc, buf):
        handles = []
        for g in range(NG):
            h = pltpu.async_copy(
                table_hbm.at[quad_v.at[pl.ds(sc * RPS + g * GCH, GCH)]],
                rows_v.at[buf].at[pl.ds(g * GCH, GCH)],
                sem,
            )
            handles.append(h)
        return handles

    iota = lax.iota(jnp.int32, LANES)
    buf_idx = [jnp.full((LANES,), k, jnp.int32) for k in range(2)]
    himask = jnp.full((LANES,), -65536, jnp.int32)

    def pool_and_store(sc, buf):
        def body_b(b, _):
            accs = [jnp.zeros((LANES,), jnp.float32) for _ in range(DV)]
            for f in range(F):
                j = b * F + f
                jsplat = jnp.full((LANES,), 0, jnp.int32) + j
                gsplat = jsplat + sc * RPS
                idxspl = plsc.load_gather(idx_v, [gsplat])
                sec = (idxspl >> 10) & 3
                wbase = (sec & 1) * D + iota
                hi_sel = (sec >> 1) == 1
                for c in range(DV):
                    w = plsc.load_gather(
                        rows_v, [buf_idx[buf], jsplat, wbase + c * LANES])
                    vbits = jnp.where(hi_sel, w & himask, w << 16)
                    accs[c] = accs[c] + plsc.bitcast(vbits, jnp.float32)
            for c in range(DV):
                pooled_v[b, pl.ds(c * LANES, LANES)] = accs[c]
            return 0

        lax.fori_loop(0, SUB, body_b, 0)
        pltpu.sync_copy(pooled_v, out_hbm.at[pl.ds(base_row + sc * SUB, SUB)])

    pending = fire(0, 0)
    for sc in range(NSUB):
        buf = sc % 2
        for h in pending:
            h.wait()
        if sc + 1 < NSUB:
            pending = fire(sc + 1, (sc + 1) % 2)
        pool_and_store(sc, buf)


@jax.jit
def _gather_pool(idx_flat, quad_flat, table_quads):
    mesh = plsc.VectorSubcoreMesh(core_axis_name="c", subcore_axis_name="s")
    kern = functools.partial(
        pl.kernel,
        out_type=jax.ShapeDtypeStruct((B, D), jnp.float32),
        mesh=mesh,
        scratch_types=[
            pltpu.VMEM((BPW * F,), jnp.int32),
            pltpu.VMEM((BPW * F,), jnp.int32),
            pltpu.VMEM((2, RPS, PD), jnp.int32),
            pltpu.VMEM((SUB, D), jnp.float32),
            pltpu.SemaphoreType.DMA,
        ],
        compiler_params=pltpu.CompilerParams(needs_layout_passes=False),
    )(_pool_body)
    return kern(idx_flat, quad_flat, table_quads)


def _tr_body(t_ref, o_ref):
    xin = t_ref[...]
    x = jnp.concatenate(
        [xin[:, k * QB:(k + 1) * QB] for k in range(4)], axis=0)
    xt = x.T
    blo = xt[:, :128].astype(jnp.bfloat16)
    bhi = xt[:, 128:].astype(jnp.bfloat16)
    ulo = lax.bitcast_convert_type(blo, jnp.uint16).astype(jnp.uint32)
    uhi = lax.bitcast_convert_type(bhi, jnp.uint16).astype(jnp.uint32)
    o_ref[...] = lax.bitcast_convert_type(ulo | (uhi << 16), jnp.int32)


def _transpose_quads(tableT):
    return pl.pallas_call(
        _tr_body,
        grid=(NTB,),
        in_specs=[pl.BlockSpec((D, TCB), lambda i: (0, i))],
        out_specs=pl.BlockSpec((QB, PD), lambda i: (i, 0)),
        out_shape=jax.ShapeDtypeStruct((VP, PD), jnp.int32),
    )(tableT)


def _mlp_body(p_ref, w1_ref, b1_ref, w2_ref, b2_ref, y_ref):
    h = jnp.dot(p_ref[...], w1_ref[...], preferred_element_type=jnp.float32)
    h = jnp.maximum(h + b1_ref[...], 0.0)
    y_ref[...] = jnp.dot(h, w2_ref[...], preferred_element_type=jnp.float32) + b2_ref[...]


MLP_BLK = 1024


def _mlp(pooled, W1, b1, W2, b2):
    return pl.pallas_call(
        _mlp_body,
        grid=(B // MLP_BLK,),
        in_specs=[
            pl.BlockSpec((MLP_BLK, D), lambda i: (i, 0)),
            pl.BlockSpec((D, H), lambda i: (0, 0)),
            pl.BlockSpec((1, H), lambda i: (0, 0)),
            pl.BlockSpec((H, A), lambda i: (0, 0)),
            pl.BlockSpec((1, A), lambda i: (0, 0)),
        ],
        out_specs=pl.BlockSpec((MLP_BLK, A), lambda i: (i, 0)),
        out_shape=jax.ShapeDtypeStruct((B, A), jnp.float32),
    )(pooled, W1, b1, W2, b2)


def kernel(indices, table, W1, b1, W2, b2):
    idx_flat = indices.reshape(-1).astype(jnp.int32)
    quad_flat = ((idx_flat >> 12) << 10) | (idx_flat & (QB - 1))
    table_quads = _transpose_quads(table.T)
    pooled = _gather_pool(idx_flat, quad_flat, table_quads)
    return _mlp(pooled, W1, b1.reshape(1, H), W2, b2.reshape(1, A))


# speedup vs baseline: 3.5792x; 1.5421x over previous
import functools

import jax
import jax.numpy as jnp
from jax import lax
from jax.experimental import pallas as pl
from jax.experimental.pallas import tpu as pltpu
from jax.experimental.pallas import tpu_sc as plsc

B = 4096
F = 20
V = 1000000
D = 64
H = 512
A = 4

NC = 2
NS = 16
NW = NC * NS
BPW = B // NW
SUB = 16
NSUB = BPW // SUB
RPS = SUB * F
GCH = 64
NG = RPS // GCH
LANES = 16
DV = D // LANES
PD = 128

TCB = 16384
QB = TCB // 4
QSH = 12
NTB = -(-V // TCB)
VP = NTB * QB


def _pool_body(idx_hbm, quad_hbm, table_hbm, out_hbm, idx_v, quad_v, rows_v,
               pooled_v, sem):
    wid = lax.axis_index("s") * NC + lax.axis_index("c")
    base_row = wid * BPW

    pltpu.sync_copy(idx_hbm.at[pl.ds(base_row * F, BPW * F)], idx_v)
    pltpu.sync_copy(quad_hbm.at[pl.ds(base_row * F, BPW * F)], quad_v)

    def fire(sc, buf):
        handles = []
        for g in range(NG):
            h = pltpu.async_copy(
                table_hbm.at[quad_v.at[pl.ds(sc * RPS + g * GCH, GCH)]],
                rows_v.at[buf].at[pl.ds(g * GCH, GCH)],
                sem,
            )
            handles.append(h)
        return handles

    iota = lax.iota(jnp.int32, LANES)
    buf_idx = [jnp.full((LANES,), k, jnp.int32) for k in range(2)]
    himask = jnp.full((LANES,), -65536, jnp.int32)

    def pool_and_store(sc, buf):
        def body_b(b, _):
            accs = [jnp.zeros((LANES,), jnp.float32) for _ in range(DV)]
            for f in range(F):
                j = b * F + f
                jsplat = jnp.full((LANES,), 0, jnp.int32) + j
                gsplat = jsplat + sc * RPS
                idxspl = plsc.load_gather(idx_v, [gsplat])
                sec = (idxspl >> QSH) & 3
                wbase = (sec & 1) * D + iota
                hi_sel = (sec >> 1) == 1
                for c in range(DV):
                    w = plsc.load_gather(
                        rows_v, [buf_idx[buf], jsplat, wbase + c * LANES])
                    vbits = jnp.where(hi_sel, w & himask, w << 16)
                    accs[c] = accs[c] + plsc.bitcast(vbits, jnp.float32)
            for c in range(DV):
                pooled_v[b, pl.ds(c * LANES, LANES)] = accs[c]
            return 0

        lax.fori_loop(0, SUB, body_b, 0)
        pltpu.sync_copy(pooled_v, out_hbm.at[pl.ds(base_row + sc * SUB, SUB)])

    pending = fire(0, 0)
    for sc in range(NSUB):
        buf = sc % 2
        for h in pending:
            h.wait()
        if sc + 1 < NSUB:
            pending = fire(sc + 1, (sc + 1) % 2)
        pool_and_store(sc, buf)


@jax.jit
def _gather_pool(idx_flat, quad_flat, table_quads):
    mesh = plsc.VectorSubcoreMesh(core_axis_name="c", subcore_axis_name="s")
    kern = functools.partial(
        pl.kernel,
        out_type=jax.ShapeDtypeStruct((B, D), jnp.float32),
        mesh=mesh,
        scratch_types=[
            pltpu.VMEM((BPW * F,), jnp.int32),
            pltpu.VMEM((BPW * F,), jnp.int32),
            pltpu.VMEM((2, RPS, PD), jnp.int32),
            pltpu.VMEM((SUB, D), jnp.float32),
            pltpu.SemaphoreType.DMA,
        ],
        compiler_params=pltpu.CompilerParams(needs_layout_passes=False),
    )(_pool_body)
    return kern(idx_flat, quad_flat, table_quads)


def _tr_body(t_ref, o_ref):
    xin = t_ref[...]
    x = jnp.concatenate(
        [xin[:, k * QB:(k + 1) * QB] for k in range(4)], axis=0)
    xt = x.T
    blo = xt[:, :128].astype(jnp.bfloat16)
    bhi = xt[:, 128:].astype(jnp.bfloat16)
    ulo = lax.bitcast_convert_type(blo, jnp.uint16).astype(jnp.uint32)
    uhi = lax.bitcast_convert_type(bhi, jnp.uint16).astype(jnp.uint32)
    o_ref[...] = lax.bitcast_convert_type(ulo | (uhi << 16), jnp.int32)


def _transpose_quads(tableT):
    return pl.pallas_call(
        _tr_body,
        grid=(NTB,),
        in_specs=[pl.BlockSpec((D, TCB), lambda i: (0, i))],
        out_specs=pl.BlockSpec((QB, PD), lambda i: (i, 0)),
        out_shape=jax.ShapeDtypeStruct((VP, PD), jnp.int32),
    )(tableT)


def _mlp_body(p_ref, w1_ref, b1_ref, w2_ref, b2_ref, y_ref):
    h = jnp.dot(p_ref[...], w1_ref[...], preferred_element_type=jnp.float32)
    h = jnp.maximum(h + b1_ref[...], 0.0)
    y_ref[...] = jnp.dot(h, w2_ref[...], preferred_element_type=jnp.float32) + b2_ref[...]


MLP_BLK = 1024


def _mlp(pooled, W1, b1, W2, b2):
    return pl.pallas_call(
        _mlp_body,
        grid=(B // MLP_BLK,),
        in_specs=[
            pl.BlockSpec((MLP_BLK, D), lambda i: (i, 0)),
            pl.BlockSpec((D, H), lambda i: (0, 0)),
            pl.BlockSpec((1, H), lambda i: (0, 0)),
            pl.BlockSpec((H, A), lambda i: (0, 0)),
            pl.BlockSpec((1, A), lambda i: (0, 0)),
        ],
        out_specs=pl.BlockSpec((MLP_BLK, A), lambda i: (i, 0)),
        out_shape=jax.ShapeDtypeStruct((B, A), jnp.float32),
    )(pooled, W1, b1, W2, b2)


def kernel(indices, table, W1, b1, W2, b2):
    idx_flat = indices.reshape(-1).astype(jnp.int32)
    quad_flat = ((idx_flat >> (QSH + 2)) << QSH) | (idx_flat & (QB - 1))
    table_quads = _transpose_quads(table.T)
    pooled = _gather_pool(idx_flat, quad_flat, table_quads)
    return _mlp(pooled, W1, b1.reshape(1, H), W2, b2.reshape(1, A))
